# Initial kernel scaffold; baseline (speedup 1.0000x reference)
import jax
import jax.numpy as jnp
from jax.experimental import pallas as pl


def kernel(y_pred, s, a, b, alpha, y_true, index):
    raise NotImplementedError("write your pallas kernel here")


# speedup vs baseline: 1.7854x; 1.7854x over previous
import functools

import jax
import jax.numpy as jnp
from jax import lax
from jax.experimental import pallas as pl
from jax.experimental.pallas import tpu as pltpu
from jax.experimental.pallas import tpu_sc as plsc

_DATA_LEN = 1000000
_BATCH = 16384
_GAMMA = 0.9
_TAU = 0.1

_NC = 2
_NS = 16
_L = 16
_NW = _NC * _NS
_CHUNK = _BATCH // _NW
_SLICE = 62504
_TAB = _NS * _SLICE
_NSCAN = _BATCH // _L
_NCH = _CHUNK // _L
_NSUM = 10

_LN2 = 0.6931471805599453
_SQRT2 = 1.4142135623730951


def _ln(v):
    bits = plsc.bitcast(v, jnp.int32)
    e = lax.shift_right_logical(bits, 23) - 127
    m = plsc.bitcast(
        jnp.bitwise_or(jnp.bitwise_and(bits, 0x007FFFFF), 0x3F800000), jnp.float32)
    big = m > _SQRT2
    m = jnp.where(big, 0.5 * m, m)
    e = jnp.where(big, e + 1, e)
    t = (m - 1.0) / (m + 1.0)
    u = t * t
    poly = t * (2.0 + u * (2.0 / 3.0 + u * (2.0 / 5.0 + u * (2.0 / 7.0 + u * (2.0 / 9.0)))))
    return e.astype(jnp.float32) * _LN2 + poly


def _sc_body(yp_hbm, s_hbm, yt_hbm, idx_hbm, out_hbm, tab_hbm,
             idx_full, yp_full, table,
             idxc, idxo, ywc, svc, ypc, ytc, accs, sem):
    cid = lax.axis_index("c")
    sid = lax.axis_index("s")
    wid = sid * _NC + cid
    base_i = sid * _SLICE
    base_p = wid * _CHUNK

    c_idx = pltpu.async_copy(idx_hbm, idx_full, sem)
    c_yp = pltpu.async_copy(yp_hbm, yp_full, sem)
    c_i2 = pltpu.async_copy(idx_hbm.at[pl.ds(base_p, _CHUNK)], idxc, sem)
    c_ypc = pltpu.async_copy(yp_hbm.at[pl.ds(base_p, _CHUNK)], ypc, sem)
    c_ytc = pltpu.async_copy(yt_hbm.at[pl.ds(base_p, _CHUNK)], ytc, sem)
    c_idx.wait()
    c_yp.wait()

    def scan_step(i, carry):
        off = i * _L
        iv = idx_full[pl.ds(off, _L)]
        yv = yp_full[pl.ds(off, _L)]
        loc = iv - base_i
        m = jnp.logical_and(iv >= base_i, iv < base_i + _SLICE)
        plsc.store_scatter(table, [loc], _GAMMA * yv, mask=m)
        return carry

    lax.fori_loop(0, _NSCAN, scan_step, 0)

    pltpu.sync_copy(table, tab_hbm.at[pl.ds(cid * _TAB + base_i, _SLICE)])
    plsc.subcore_barrier()

    c_i2.wait()
    c_ypc.wait()
    c_ytc.wait()

    tab_off = cid * _TAB

    def off_step(i, carry):
        o = i * _L
        idxo[pl.ds(o, _L)] = idxc[pl.ds(o, _L)] + tab_off
        return carry

    lax.fori_loop(0, _NCH, off_step, 0)

    gathers = []
    for j in range(_CHUNK // 128):
        isl = idxc.at[pl.ds(j * 128, 128)]
        osl = idxo.at[pl.ds(j * 128, 128)]
        gathers.append(pltpu.async_copy(
            tab_hbm.at[osl], ywc.at[pl.ds(j * 128, 128)], sem))
        gathers.append(pltpu.async_copy(
            s_hbm.at[isl], svc.at[pl.ds(j * 128, 128)], sem))
    for g in gathers:
        g.wait()

    zero = jnp.zeros((_L,), jnp.float32)

    def chunk_step(i, acc):
        anp, ann, s1p, s1n, s2p, s2n, s3p, s3n, slp, sln = acc
        off = i * _L
        ypv = ypc[pl.ds(off, _L)]
        ytv = ytc[pl.ds(off, _L)]
        ywv = ywc[pl.ds(off, _L)]
        svv = svc[pl.ds(off, _L)]
        vs = (1.0 - _GAMMA) * svv + ywv
        logs = _TAU * _ln(vs)
        gw = ypv / vs
        mp = jnp.where(ytv == 1, 1.0, 0.0)
        mn = jnp.where(ytv == 0, 1.0, 0.0)
        lg = logs * gw
        l2 = logs * logs
        return (anp + mp, ann + mn, s1p + mp * lg, s1n + mn * lg,
                s2p + mp * gw, s2n + mn * gw, s3p + mp * l2, s3n + mn * l2,
                slp + mp * logs, sln + mn * logs)

    accf = lax.fori_loop(0, _NCH, chunk_step, (zero,) * _NSUM)
    for k in range(_NSUM):
        accs[pl.ds(k * _L, _L)] = accf[k]
    for k in range(_NSUM):
        pltpu.sync_copy(accs.at[pl.ds(k * _L, _L)],
                        out_hbm.at[k, pl.ds(wid * _L, _L)])


def _combine_body(p_ref, a_ref, b_ref, al_ref, o_ref):
    rs = jnp.sum(p_ref[...], axis=1, keepdims=True)
    np_ = rs[0:1, :]
    nn_ = rs[1:2, :]
    s1p = rs[2:3, :]
    s1n = rs[3:4, :]
    s2p = rs[4:5, :]
    s2n = rs[5:6, :]
    s3p = rs[6:7, :]
    s3n = rs[7:8, :]
    slp = rs[8:9, :]
    sln = rs[9:10, :]
    a = a_ref[...].reshape(1, 1)
    b = b_ref[...].reshape(1, 1)
    al = al_ref[...].reshape(1, 1)
    loss = (2.0 * _TAU * (s1p - a * s2p) / np_
            + 2.0 * _TAU * (s1n - b * s2n) / nn_
            + al * _TAU * (s2n / nn_ - s2p / np_)
            + (s3p - 2.0 * a * slp + a * a * np_) / np_
            + (s3n - 2.0 * b * sln + b * b * nn_) / nn_)
    o_ref[...] = loss


def kernel(y_pred, s, a, b, alpha, y_true, index):
    yp1 = jnp.reshape(y_pred, (_BATCH,))
    s1 = jnp.reshape(s, (_DATA_LEN,))
    yt1 = jnp.reshape(y_true, (_BATCH,))
    idx1 = jnp.reshape(index, (_BATCH,))

    mesh = plsc.VectorSubcoreMesh(
        core_axis_name="c", subcore_axis_name="s", num_cores=_NC, num_subcores=_NS)
    partials, _ = pl.kernel(
        _sc_body,
        out_type=(jax.ShapeDtypeStruct((_NSUM, _NW * _L), jnp.float32),
                  jax.ShapeDtypeStruct((_NC * _TAB,), jnp.float32)),
        mesh=mesh,
        compiler_params=pltpu.CompilerParams(needs_layout_passes=False),
        scratch_types=[
            pltpu.VMEM((_BATCH,), jnp.int32),
            pltpu.VMEM((_BATCH,), jnp.float32),
            pltpu.VMEM((_SLICE,), jnp.float32),
            pltpu.VMEM((_CHUNK,), jnp.int32),
            pltpu.VMEM((_CHUNK,), jnp.int32),
            pltpu.VMEM((_CHUNK,), jnp.float32),
            pltpu.VMEM((_CHUNK,), jnp.float32),
            pltpu.VMEM((_CHUNK,), jnp.float32),
            pltpu.VMEM((_CHUNK,), jnp.int32),
            pltpu.VMEM((_NSUM * _L,), jnp.float32),
            pltpu.SemaphoreType.DMA,
        ],
    )(yp1, s1, yt1, idx1)

    loss = pl.pallas_call(
        _combine_body,
        out_shape=jax.ShapeDtypeStruct((1, 1), jnp.float32),
    )(partials, a, b, alpha)
    return jnp.reshape(loss, (1,))
